# baseline (speedup 1.0000x reference)
import functools

import jax
import jax.numpy as jnp
from jax import lax
from jax.experimental import pallas as pl
from jax.experimental.pallas import tpu as pltpu
from jax.experimental.pallas import tpu_sc as plsc

_NC = 2
_NS = 16
_EBLK = 192
_G = 128


def _enc_body(x_ref, w_ref, b_ref, lo_ref, hi_ref):
    h = jnp.dot(x_ref[...], w_ref[...], preferred_element_type=jnp.float32)
    h = h + b_ref[...]
    dh = lo_ref.shape[1]
    lo_ref[...] = h[:, :dh]
    hi_ref[...] = h[:, dh:]


def _encoder(x, w, b):
    n = x.shape[0]
    d = w.shape[1]
    half = jax.ShapeDtypeStruct((n, d // 2), jnp.float32)
    return pl.pallas_call(
        _enc_body,
        out_shape=(half, half),
    )(x, w, b)


def _mlp_bn_body(hlo_ref, hhi_ref, agg_ref, eps_ref, w1_ref, b1_ref, w2_ref,
                 b2_ref, gamma_ref, beta_ref, lo_ref, hi_ref, *, relu):
    n = hlo_ref.shape[0]
    h = jnp.concatenate([hlo_ref[...], hhi_ref[...]], axis=1)
    agg = jnp.concatenate([agg_ref[pl.ds(0, n), :], agg_ref[pl.ds(n, n), :]],
                          axis=1)
    z = (1.0 + eps_ref[0, 0]) * h + agg
    z = jnp.dot(z, w1_ref[...], preferred_element_type=jnp.float32) + b1_ref[...]
    z = jnp.maximum(z, 0.0)
    g = jnp.dot(z, w2_ref[...], preferred_element_type=jnp.float32) + b2_ref[...]
    mu = jnp.mean(g, axis=0, keepdims=True)
    var = jnp.mean(jnp.square(g - mu), axis=0, keepdims=True)
    o = gamma_ref[...] * (g - mu) * lax.rsqrt(var + 1e-5) + beta_ref[...]
    if relu:
        o = jnp.maximum(o, 0.0)
    dh = lo_ref.shape[1]
    lo_ref[...] = o[:, :dh]
    hi_ref[...] = o[:, dh:]


def _mlp_bn(hlo, hhi, aggf, eps, w1, b1, w2, b2, gamma, beta):
    n, dh = hlo.shape
    half = jax.ShapeDtypeStruct((n, dh), jnp.float32)
    return pl.pallas_call(
        functools.partial(_mlp_bn_body, relu=True),
        out_shape=(half, half),
    )(hlo, hhi, aggf, eps, w1, b1, w2, b2, gamma, beta)


def _mlp_bn_pool_body(hlo_ref, hhi_ref, agg_ref, eps_ref, w1_ref, b1_ref,
                      w2_ref, b2_ref, gamma_ref, beta_ref, batch_ref,
                      hg_ref, vg_ref):
    n = hlo_ref.shape[0]
    h = jnp.concatenate([hlo_ref[...], hhi_ref[...]], axis=1)
    agg = jnp.concatenate([agg_ref[pl.ds(0, n), :], agg_ref[pl.ds(n, n), :]],
                          axis=1)
    z = (1.0 + eps_ref[0, 0]) * h + agg
    z = jnp.dot(z, w1_ref[...], preferred_element_type=jnp.float32) + b1_ref[...]
    z = jnp.maximum(z, 0.0)
    g = jnp.dot(z, w2_ref[...], preferred_element_type=jnp.float32) + b2_ref[...]
    mu = jnp.mean(g, axis=0, keepdims=True)
    var = jnp.mean(jnp.square(g - mu), axis=0, keepdims=True)
    h2 = gamma_ref[...] * (g - mu) * lax.rsqrt(var + 1e-5) + beta_ref[...]
    seg = lax.broadcasted_iota(jnp.int32, (_G, n), 0)
    onehot = (batch_ref[...] == seg).astype(jnp.float32)
    hg_ref[...] = jnp.dot(onehot, h2, preferred_element_type=jnp.float32)
    vg_ref[...] = jnp.dot(onehot, h, preferred_element_type=jnp.float32)


def _mlp_bn_pool(hlo, hhi, aggf, eps, w1, b1, w2, b2, gamma, beta, batch):
    d = 2 * hlo.shape[1]
    return pl.pallas_call(
        _mlp_bn_pool_body,
        out_shape=(
            jax.ShapeDtypeStruct((_G, d), jnp.float32),
            jax.ShapeDtypeStruct((_G, d), jnp.float32),
        ),
    )(hlo, hhi, aggf, eps, w1, b1, w2, b2, gamma, beta, batch)


def _make_agg(n, e, dh):
    rows_per_tile = (n // (8 * _NS)) * 8
    tail = n - _NS * rows_per_tile
    ept = e // _NS
    chunks = ept // _EBLK
    etail = ept - chunks * _EBLK
    mesh = plsc.VectorSubcoreMesh(core_axis_name="c", subcore_axis_name="s")

    @functools.partial(
        pl.kernel,
        out_type=jax.ShapeDtypeStruct((2 * n, dh), jnp.float32),
        mesh=mesh,
        scratch_types=[
            pltpu.VMEM((_EBLK,), jnp.int32),
            pltpu.VMEM((_EBLK,), jnp.int32),
            pltpu.VMEM((_EBLK,), jnp.int32),
            pltpu.VMEM((_EBLK,), jnp.int32),
            pltpu.VMEM((_EBLK, dh), jnp.float32),
            pltpu.VMEM((_EBLK, dh), jnp.float32),
            pltpu.VMEM((max(etail, 8),), jnp.int32),
            pltpu.VMEM((max(etail, 8),), jnp.int32),
            pltpu.VMEM_SHARED((n, dh), jnp.float32),
            pltpu.SemaphoreType.DMA,
            pltpu.SemaphoreType.DMA,
            pltpu.SemaphoreType.DMA,
            pltpu.SemaphoreType.DMA,
        ],
    )
    def agg(h_lo, h_hi, edges, zeros, out, idx_s0, idx_s1, idx_d0, idx_d1,
            rows0, rows1, tidx_s, tidx_d, acc, sem0, sem1, sem_i0, sem_i1):
        src = edges.at[pl.ds(0, e)]
        dst = edges.at[pl.ds(e, e)]
        c = lax.axis_index("c")
        s = lax.axis_index("s")
        idx_s = (idx_s0, idx_s1)
        idx_d = (idx_d0, idx_d1)
        rows = (rows0, rows1)
        sems = (sem0, sem1)
        sems_i = (sem_i0, sem_i1)
        ebase = s * ept

        def load_idx(k, b):
            pltpu.async_copy(src.at[pl.ds(ebase + k * _EBLK, _EBLK)],
                             idx_s[b], sems_i[b])
            pltpu.async_copy(dst.at[pl.ds(ebase + k * _EBLK, _EBLK)],
                             idx_d[b], sems_i[b])

        def wait_idx(b):
            pltpu.make_async_copy(src.at[pl.ds(0, _EBLK)], idx_s[b],
                                  sems_i[b]).wait()
            pltpu.make_async_copy(dst.at[pl.ds(0, _EBLK)], idx_d[b],
                                  sems_i[b]).wait()

        def fire_gather(b):
            @pl.when(c == 0)
            def _():
                pltpu.async_copy(h_lo.at[idx_s[b]], rows[b], sems[b])

            @pl.when(c == 1)
            def _():
                pltpu.async_copy(h_hi.at[idx_s[b]], rows[b], sems[b])

        load_idx(0, 0)
        pltpu.sync_copy(zeros.at[pl.ds(0, rows_per_tile)],
                        acc.at[pl.ds(s * rows_per_tile, rows_per_tile)])
        if tail:
            @pl.when(s == _NS - 1)
            def _():
                pltpu.sync_copy(zeros.at[pl.ds(0, tail)],
                                acc.at[pl.ds(_NS * rows_per_tile, tail)])
        wait_idx(0)
        fire_gather(0)
        if chunks > 1:
            load_idx(1, 1)
        plsc.subcore_barrier()

        def pair_body(i, carry):
            for b in (0, 1):
                k = 2 * i + b
                pltpu.make_async_copy(h_lo.at[idx_s[b]], rows[b],
                                      sems[b]).wait()

                @pl.when(k + 1 < chunks)
                def _():
                    wait_idx(1 - b)
                    fire_gather(1 - b)

                pltpu.sync_copy(rows[b], acc.at[idx_d[b]], add=True)

                @pl.when(k + 2 < chunks)
                def _():
                    load_idx(k + 2, b)
            return carry

        lax.fori_loop(0, chunks // 2, pair_body, 0)
        if chunks % 2:
            b = (chunks - 1) % 2
            pltpu.make_async_copy(h_lo.at[idx_s[b]], rows[b],
                                  sems[b]).wait()
            pltpu.sync_copy(rows[b], acc.at[idx_d[b]], add=True)

        if etail:
            tbase = ebase + chunks * _EBLK
            pltpu.sync_copy(src.at[pl.ds(tbase, etail)], tidx_s.at[pl.ds(0, etail)])
            pltpu.sync_copy(dst.at[pl.ds(tbase, etail)], tidx_d.at[pl.ds(0, etail)])

            @pl.when(c == 0)
            def _():
                pltpu.async_copy(h_lo.at[tidx_s], rows0.at[pl.ds(0, max(etail, 8))],
                                 sem0).wait()

            @pl.when(c == 1)
            def _():
                pltpu.async_copy(h_hi.at[tidx_s], rows0.at[pl.ds(0, max(etail, 8))],
                                 sem1).wait()

            pltpu.sync_copy(rows0.at[pl.ds(0, max(etail, 8))], acc.at[tidx_d],
                            add=True)
        plsc.subcore_barrier()
        pltpu.sync_copy(
            acc.at[pl.ds(s * rows_per_tile, rows_per_tile)],
            out.at[pl.ds(c * n + s * rows_per_tile, rows_per_tile)],
        )
        if tail:
            @pl.when(s == _NS - 1)
            def _():
                pltpu.sync_copy(
                    acc.at[pl.ds(_NS * rows_per_tile, tail)],
                    out.at[pl.ds(c * n + _NS * rows_per_tile, tail)],
                )

    return agg


def kernel(x, edge_index, batch, W_enc, b_enc, eps1, W1_1, b1_1, W2_1, b2_1,
           gamma1, beta1, eps2, W1_2, b1_2, W2_2, b2_2, gamma2, beta2):
    n, f = x.shape
    d = W_enc.shape[1]
    dh = d // 2
    e = edge_index.shape[1]

    zeros = jnp.zeros((n // _NS, dh), jnp.float32)
    eps1_ = jnp.reshape(eps1, (1, 1))
    eps2_ = jnp.reshape(eps2, (1, 1))
    b_enc_ = jnp.reshape(b_enc, (1, d))
    b1_1_ = jnp.reshape(b1_1, (1, 2 * d))
    b2_1_ = jnp.reshape(b2_1, (1, d))
    gamma1_ = jnp.reshape(gamma1, (1, d))
    beta1_ = jnp.reshape(beta1, (1, d))
    b1_2_ = jnp.reshape(b1_2, (1, 2 * d))
    b2_2_ = jnp.reshape(b2_2, (1, d))
    gamma2_ = jnp.reshape(gamma2, (1, d))
    beta2_ = jnp.reshape(beta2, (1, d))
    batch_ = jnp.reshape(batch, (1, n))

    agg_fn = _make_agg(n, e, dh)

    edges_flat = jnp.ravel(edge_index)

    h0lo, h0hi = _encoder(x, W_enc, b_enc_)
    agg1 = agg_fn(h0lo, h0hi, edges_flat, zeros)
    h1lo, h1hi = _mlp_bn(h0lo, h0hi, agg1, eps1_, W1_1, b1_1_, W2_1, b2_1_,
                         gamma1_, beta1_)
    agg2 = agg_fn(h1lo, h1hi, edges_flat, zeros)
    hg, vg = _mlp_bn_pool(h1lo, h1hi, agg2, eps2_, W1_2, b1_2_, W2_2, b2_2_,
                          gamma2_, beta2_, batch_)
    return (hg, vg)


# speedup vs baseline: 1.0080x; 1.0080x over previous
import functools

import jax
import jax.numpy as jnp
from jax import lax
from jax.experimental import pallas as pl
from jax.experimental.pallas import tpu as pltpu
from jax.experimental.pallas import tpu_sc as plsc

_NC = 2
_NS = 16
_EBLK = 192
_G = 128


def _enc_body(x_ref, w_ref, b_ref, lo_ref, hi_ref):
    h = jnp.dot(x_ref[...], w_ref[...], preferred_element_type=jnp.float32)
    h = h + b_ref[...]
    dh = lo_ref.shape[1]
    lo_ref[...] = h[:, :dh]
    hi_ref[...] = h[:, dh:]


def _encoder(x, w, b):
    n, f = x.shape
    d = w.shape[1]
    blk = n // 10
    half = jax.ShapeDtypeStruct((n, d // 2), jnp.float32)
    return pl.pallas_call(
        _enc_body,
        grid=(10,),
        in_specs=[
            pl.BlockSpec((blk, f), lambda i: (i, 0)),
            pl.BlockSpec((f, d), lambda i: (0, 0)),
            pl.BlockSpec((1, d), lambda i: (0, 0)),
        ],
        out_specs=(
            pl.BlockSpec((blk, d // 2), lambda i: (i, 0)),
            pl.BlockSpec((blk, d // 2), lambda i: (i, 0)),
        ),
        out_shape=(half, half),
    )(x, w, b)


def _mlp_bn_body(hlo_ref, hhi_ref, agg_ref, eps_ref, w1_ref, b1_ref, w2_ref,
                 b2_ref, gamma_ref, beta_ref, lo_ref, hi_ref, *, relu):
    n = hlo_ref.shape[0]
    h = jnp.concatenate([hlo_ref[...], hhi_ref[...]], axis=1)
    agg = jnp.concatenate([agg_ref[pl.ds(0, n), :], agg_ref[pl.ds(n, n), :]],
                          axis=1)
    z = (1.0 + eps_ref[0, 0]) * h + agg
    z = jnp.dot(z, w1_ref[...], preferred_element_type=jnp.float32) + b1_ref[...]
    z = jnp.maximum(z, 0.0)
    g = jnp.dot(z, w2_ref[...], preferred_element_type=jnp.float32) + b2_ref[...]
    mu = jnp.mean(g, axis=0, keepdims=True)
    var = jnp.mean(jnp.square(g - mu), axis=0, keepdims=True)
    o = gamma_ref[...] * (g - mu) * lax.rsqrt(var + 1e-5) + beta_ref[...]
    if relu:
        o = jnp.maximum(o, 0.0)
    dh = lo_ref.shape[1]
    lo_ref[...] = o[:, :dh]
    hi_ref[...] = o[:, dh:]


def _mlp_bn(hlo, hhi, aggf, eps, w1, b1, w2, b2, gamma, beta):
    n, dh = hlo.shape
    half = jax.ShapeDtypeStruct((n, dh), jnp.float32)
    return pl.pallas_call(
        functools.partial(_mlp_bn_body, relu=True),
        out_shape=(half, half),
    )(hlo, hhi, aggf, eps, w1, b1, w2, b2, gamma, beta)


def _mlp_bn_pool_body(hlo_ref, hhi_ref, agg_ref, eps_ref, w1_ref, b1_ref,
                      w2_ref, b2_ref, gamma_ref, beta_ref, batch_ref,
                      hg_ref, vg_ref):
    n = hlo_ref.shape[0]
    h = jnp.concatenate([hlo_ref[...], hhi_ref[...]], axis=1)
    agg = jnp.concatenate([agg_ref[pl.ds(0, n), :], agg_ref[pl.ds(n, n), :]],
                          axis=1)
    z = (1.0 + eps_ref[0, 0]) * h + agg
    z = jnp.dot(z, w1_ref[...], preferred_element_type=jnp.float32) + b1_ref[...]
    z = jnp.maximum(z, 0.0)
    g = jnp.dot(z, w2_ref[...], preferred_element_type=jnp.float32) + b2_ref[...]
    mu = jnp.mean(g, axis=0, keepdims=True)
    var = jnp.mean(jnp.square(g - mu), axis=0, keepdims=True)
    h2 = gamma_ref[...] * (g - mu) * lax.rsqrt(var + 1e-5) + beta_ref[...]
    seg = lax.broadcasted_iota(jnp.int32, (_G, n), 0)
    onehot = (batch_ref[...] == seg).astype(jnp.float32)
    hg_ref[...] = jnp.dot(onehot, h2, preferred_element_type=jnp.float32)
    vg_ref[...] = jnp.dot(onehot, h, preferred_element_type=jnp.float32)


def _mlp_bn_pool(hlo, hhi, aggf, eps, w1, b1, w2, b2, gamma, beta, batch):
    d = 2 * hlo.shape[1]
    return pl.pallas_call(
        _mlp_bn_pool_body,
        out_shape=(
            jax.ShapeDtypeStruct((_G, d), jnp.float32),
            jax.ShapeDtypeStruct((_G, d), jnp.float32),
        ),
    )(hlo, hhi, aggf, eps, w1, b1, w2, b2, gamma, beta, batch)


def _make_agg(n, e, dh):
    rows_per_tile = (n // (8 * _NS)) * 8
    tail = n - _NS * rows_per_tile
    ept = e // _NS
    chunks = ept // _EBLK
    etail = ept - chunks * _EBLK
    mesh = plsc.VectorSubcoreMesh(core_axis_name="c", subcore_axis_name="s")

    @functools.partial(
        pl.kernel,
        out_type=jax.ShapeDtypeStruct((2 * n, dh), jnp.float32),
        mesh=mesh,
        scratch_types=[
            pltpu.VMEM((_EBLK,), jnp.int32),
            pltpu.VMEM((_EBLK,), jnp.int32),
            pltpu.VMEM((_EBLK,), jnp.int32),
            pltpu.VMEM((_EBLK,), jnp.int32),
            pltpu.VMEM((_EBLK, dh), jnp.float32),
            pltpu.VMEM((_EBLK, dh), jnp.float32),
            pltpu.VMEM((max(etail, 8),), jnp.int32),
            pltpu.VMEM((max(etail, 8),), jnp.int32),
            pltpu.VMEM_SHARED((n, dh), jnp.float32),
            pltpu.SemaphoreType.DMA,
            pltpu.SemaphoreType.DMA,
            pltpu.SemaphoreType.DMA,
            pltpu.SemaphoreType.DMA,
        ],
    )
    def agg(h_lo, h_hi, edges, zeros, out, idx_s0, idx_s1, idx_d0, idx_d1,
            rows0, rows1, tidx_s, tidx_d, acc, sem0, sem1, sem_i0, sem_i1):
        src = edges.at[pl.ds(0, e)]
        dst = edges.at[pl.ds(e, e)]
        c = lax.axis_index("c")
        s = lax.axis_index("s")
        idx_s = (idx_s0, idx_s1)
        idx_d = (idx_d0, idx_d1)
        rows = (rows0, rows1)
        sems = (sem0, sem1)
        sems_i = (sem_i0, sem_i1)
        ebase = s * ept

        def load_idx(k, b):
            pltpu.async_copy(src.at[pl.ds(ebase + k * _EBLK, _EBLK)],
                             idx_s[b], sems_i[b])
            pltpu.async_copy(dst.at[pl.ds(ebase + k * _EBLK, _EBLK)],
                             idx_d[b], sems_i[b])

        def wait_idx(b):
            pltpu.make_async_copy(src.at[pl.ds(0, _EBLK)], idx_s[b],
                                  sems_i[b]).wait()
            pltpu.make_async_copy(dst.at[pl.ds(0, _EBLK)], idx_d[b],
                                  sems_i[b]).wait()

        def fire_gather(b):
            @pl.when(c == 0)
            def _():
                pltpu.async_copy(h_lo.at[idx_s[b]], rows[b], sems[b])

            @pl.when(c == 1)
            def _():
                pltpu.async_copy(h_hi.at[idx_s[b]], rows[b], sems[b])

        load_idx(0, 0)
        pltpu.sync_copy(zeros.at[pl.ds(0, rows_per_tile)],
                        acc.at[pl.ds(s * rows_per_tile, rows_per_tile)])
        if tail:
            @pl.when(s == _NS - 1)
            def _():
                pltpu.sync_copy(zeros.at[pl.ds(0, tail)],
                                acc.at[pl.ds(_NS * rows_per_tile, tail)])
        wait_idx(0)
        fire_gather(0)
        if chunks > 1:
            load_idx(1, 1)
        plsc.subcore_barrier()

        def pair_body(i, carry):
            for b in (0, 1):
                k = 2 * i + b
                pltpu.make_async_copy(h_lo.at[idx_s[b]], rows[b],
                                      sems[b]).wait()

                @pl.when(k + 1 < chunks)
                def _():
                    wait_idx(1 - b)
                    fire_gather(1 - b)

                pltpu.sync_copy(rows[b], acc.at[idx_d[b]], add=True)

                @pl.when(k + 2 < chunks)
                def _():
                    load_idx(k + 2, b)
            return carry

        lax.fori_loop(0, chunks // 2, pair_body, 0)
        if chunks % 2:
            b = (chunks - 1) % 2
            pltpu.make_async_copy(h_lo.at[idx_s[b]], rows[b],
                                  sems[b]).wait()
            pltpu.sync_copy(rows[b], acc.at[idx_d[b]], add=True)

        if etail:
            tbase = ebase + chunks * _EBLK
            pltpu.sync_copy(src.at[pl.ds(tbase, etail)], tidx_s.at[pl.ds(0, etail)])
            pltpu.sync_copy(dst.at[pl.ds(tbase, etail)], tidx_d.at[pl.ds(0, etail)])

            @pl.when(c == 0)
            def _():
                pltpu.async_copy(h_lo.at[tidx_s], rows0.at[pl.ds(0, max(etail, 8))],
                                 sem0).wait()

            @pl.when(c == 1)
            def _():
                pltpu.async_copy(h_hi.at[tidx_s], rows0.at[pl.ds(0, max(etail, 8))],
                                 sem1).wait()

            pltpu.sync_copy(rows0.at[pl.ds(0, max(etail, 8))], acc.at[tidx_d],
                            add=True)
        plsc.subcore_barrier()
        pltpu.sync_copy(
            acc.at[pl.ds(s * rows_per_tile, rows_per_tile)],
            out.at[pl.ds(c * n + s * rows_per_tile, rows_per_tile)],
        )
        if tail:
            @pl.when(s == _NS - 1)
            def _():
                pltpu.sync_copy(
                    acc.at[pl.ds(_NS * rows_per_tile, tail)],
                    out.at[pl.ds(c * n + _NS * rows_per_tile, tail)],
                )

    return agg


def kernel(x, edge_index, batch, W_enc, b_enc, eps1, W1_1, b1_1, W2_1, b2_1,
           gamma1, beta1, eps2, W1_2, b1_2, W2_2, b2_2, gamma2, beta2):
    n, f = x.shape
    d = W_enc.shape[1]
    dh = d // 2
    e = edge_index.shape[1]

    zeros = jnp.zeros((n // _NS, dh), jnp.float32)
    eps1_ = jnp.reshape(eps1, (1, 1))
    eps2_ = jnp.reshape(eps2, (1, 1))
    b_enc_ = jnp.reshape(b_enc, (1, d))
    b1_1_ = jnp.reshape(b1_1, (1, 2 * d))
    b2_1_ = jnp.reshape(b2_1, (1, d))
    gamma1_ = jnp.reshape(gamma1, (1, d))
    beta1_ = jnp.reshape(beta1, (1, d))
    b1_2_ = jnp.reshape(b1_2, (1, 2 * d))
    b2_2_ = jnp.reshape(b2_2, (1, d))
    gamma2_ = jnp.reshape(gamma2, (1, d))
    beta2_ = jnp.reshape(beta2, (1, d))
    batch_ = jnp.reshape(batch, (1, n))

    agg_fn = _make_agg(n, e, dh)

    edges_flat = jnp.ravel(edge_index)

    h0lo, h0hi = _encoder(x, W_enc, b_enc_)
    agg1 = agg_fn(h0lo, h0hi, edges_flat, zeros)
    h1lo, h1hi = _mlp_bn(h0lo, h0hi, agg1, eps1_, W1_1, b1_1_, W2_1, b2_1_,
                         gamma1_, beta1_)
    agg2 = agg_fn(h1lo, h1hi, edges_flat, zeros)
    hg, vg = _mlp_bn_pool(h1lo, h1hi, agg2, eps2_, W1_2, b1_2_, W2_2, b2_2_,
                          gamma2_, beta2_, batch_)
    return (hg, vg)


# speedup vs baseline: 1.0222x; 1.0141x over previous
import functools

import jax
import jax.numpy as jnp
from jax import lax
from jax.experimental import pallas as pl
from jax.experimental.pallas import tpu as pltpu
from jax.experimental.pallas import tpu_sc as plsc

_NC = 2
_NS = 16
_EBLK = 128
_NBUF = 3
_G = 128


def _enc_body(x_ref, w_ref, b_ref, lo_ref, hi_ref):
    h = jnp.dot(x_ref[...], w_ref[...], preferred_element_type=jnp.float32)
    h = h + b_ref[...]
    dh = lo_ref.shape[1]
    lo_ref[...] = h[:, :dh]
    hi_ref[...] = h[:, dh:]


def _encoder(x, w, b):
    n, f = x.shape
    d = w.shape[1]
    half = jax.ShapeDtypeStruct((n, d // 2), jnp.float32)
    return pl.pallas_call(
        _enc_body,
        out_shape=(half, half),
    )(x, w, b)


def _mlp_bn_body(hlo_ref, hhi_ref, agg_ref, eps_ref, w1_ref, b1_ref, w2_ref,
                 b2_ref, gamma_ref, beta_ref, lo_ref, hi_ref, *, relu):
    n = hlo_ref.shape[0]
    h = jnp.concatenate([hlo_ref[...], hhi_ref[...]], axis=1)
    agg = jnp.concatenate([agg_ref[pl.ds(0, n), :], agg_ref[pl.ds(n, n), :]],
                          axis=1)
    z = (1.0 + eps_ref[0, 0]) * h + agg
    z = jnp.dot(z, w1_ref[...], preferred_element_type=jnp.float32) + b1_ref[...]
    z = jnp.maximum(z, 0.0)
    g = jnp.dot(z, w2_ref[...], preferred_element_type=jnp.float32) + b2_ref[...]
    mu = jnp.mean(g, axis=0, keepdims=True)
    var = jnp.mean(jnp.square(g - mu), axis=0, keepdims=True)
    o = gamma_ref[...] * (g - mu) * lax.rsqrt(var + 1e-5) + beta_ref[...]
    if relu:
        o = jnp.maximum(o, 0.0)
    dh = lo_ref.shape[1]
    lo_ref[...] = o[:, :dh]
    hi_ref[...] = o[:, dh:]


def _mlp_bn(hlo, hhi, aggf, eps, w1, b1, w2, b2, gamma, beta):
    n, dh = hlo.shape
    half = jax.ShapeDtypeStruct((n, dh), jnp.float32)
    return pl.pallas_call(
        functools.partial(_mlp_bn_body, relu=True),
        out_shape=(half, half),
    )(hlo, hhi, aggf, eps, w1, b1, w2, b2, gamma, beta)


def _mlp_bn_pool_body(hlo_ref, hhi_ref, agg_ref, eps_ref, w1_ref, b1_ref,
                      w2_ref, b2_ref, gamma_ref, beta_ref, batch_ref,
                      hg_ref, vg_ref):
    n = hlo_ref.shape[0]
    h = jnp.concatenate([hlo_ref[...], hhi_ref[...]], axis=1)
    agg = jnp.concatenate([agg_ref[pl.ds(0, n), :], agg_ref[pl.ds(n, n), :]],
                          axis=1)
    z = (1.0 + eps_ref[0, 0]) * h + agg
    z = jnp.dot(z, w1_ref[...], preferred_element_type=jnp.float32) + b1_ref[...]
    z = jnp.maximum(z, 0.0)
    g = jnp.dot(z, w2_ref[...], preferred_element_type=jnp.float32) + b2_ref[...]
    mu = jnp.mean(g, axis=0, keepdims=True)
    var = jnp.mean(jnp.square(g - mu), axis=0, keepdims=True)
    h2 = gamma_ref[...] * (g - mu) * lax.rsqrt(var + 1e-5) + beta_ref[...]
    seg = lax.broadcasted_iota(jnp.int32, (_G, n), 0)
    onehot = (batch_ref[...] == seg).astype(jnp.float32)
    hg_ref[...] = jnp.dot(onehot, h2, preferred_element_type=jnp.float32)
    vg_ref[...] = jnp.dot(onehot, h, preferred_element_type=jnp.float32)


def _mlp_bn_pool(hlo, hhi, aggf, eps, w1, b1, w2, b2, gamma, beta, batch):
    d = 2 * hlo.shape[1]
    return pl.pallas_call(
        _mlp_bn_pool_body,
        out_shape=(
            jax.ShapeDtypeStruct((_G, d), jnp.float32),
            jax.ShapeDtypeStruct((_G, d), jnp.float32),
        ),
    )(hlo, hhi, aggf, eps, w1, b1, w2, b2, gamma, beta, batch)


def _make_agg(n, e, dh):
    rows_per_tile = (n // (8 * _NS)) * 8
    tail = n - _NS * rows_per_tile
    ept = e // _NS
    chunks = ept // _EBLK
    etail = ept - chunks * _EBLK
    mesh = plsc.VectorSubcoreMesh(core_axis_name="c", subcore_axis_name="s")

    @functools.partial(
        pl.kernel,
        out_type=jax.ShapeDtypeStruct((2 * n, dh), jnp.float32),
        mesh=mesh,
        scratch_types=(
            [pltpu.VMEM((_EBLK,), jnp.int32)] * (2 * _NBUF)
            + [pltpu.VMEM((_EBLK, dh), jnp.float32)] * _NBUF
            + [
                pltpu.VMEM((max(etail, 8),), jnp.int32),
                pltpu.VMEM((max(etail, 8),), jnp.int32),
                pltpu.VMEM_SHARED((n, dh), jnp.float32),
            ]
            + [pltpu.SemaphoreType.DMA] * (2 * _NBUF)
        ),
    )
    def agg(h_lo, h_hi, edges, zeros, out, idx_s0, idx_s1, idx_s2,
            idx_d0, idx_d1, idx_d2, rows0, rows1, rows2, tidx_s, tidx_d,
            acc, sem0, sem1, sem2, sem_i0, sem_i1, sem_i2):
        src = edges.at[pl.ds(0, e)]
        dst = edges.at[pl.ds(e, e)]
        c = lax.axis_index("c")
        s = lax.axis_index("s")
        idx_s = (idx_s0, idx_s1, idx_s2)
        idx_d = (idx_d0, idx_d1, idx_d2)
        rows = (rows0, rows1, rows2)
        sems = (sem0, sem1, sem2)
        sems_i = (sem_i0, sem_i1, sem_i2)
        ebase = s * ept

        def load_idx(k, b):
            pltpu.async_copy(src.at[pl.ds(ebase + k * _EBLK, _EBLK)],
                             idx_s[b], sems_i[b])
            pltpu.async_copy(dst.at[pl.ds(ebase + k * _EBLK, _EBLK)],
                             idx_d[b], sems_i[b])

        def wait_idx(b):
            pltpu.make_async_copy(src.at[pl.ds(0, _EBLK)], idx_s[b],
                                  sems_i[b]).wait()
            pltpu.make_async_copy(dst.at[pl.ds(0, _EBLK)], idx_d[b],
                                  sems_i[b]).wait()

        def fire_gather(b):
            @pl.when(c == 0)
            def _():
                pltpu.async_copy(h_lo.at[idx_s[b]], rows[b], sems[b])

            @pl.when(c == 1)
            def _():
                pltpu.async_copy(h_hi.at[idx_s[b]], rows[b], sems[b])

        load_idx(0, 0)
        load_idx(1, 1)
        pltpu.sync_copy(zeros.at[pl.ds(0, rows_per_tile)],
                        acc.at[pl.ds(s * rows_per_tile, rows_per_tile)])
        if tail:
            @pl.when(s == _NS - 1)
            def _():
                pltpu.sync_copy(zeros.at[pl.ds(0, tail)],
                                acc.at[pl.ds(_NS * rows_per_tile, tail)])
        wait_idx(0)
        fire_gather(0)
        if chunks > 1:
            wait_idx(1)
            fire_gather(1)
        if chunks > 2:
            load_idx(2, 2)
        plsc.subcore_barrier()

        def tri_body(i, carry):
            for b in range(_NBUF):
                k = _NBUF * i + b
                pltpu.make_async_copy(h_lo.at[idx_s[b]], rows[b],
                                      sems[b]).wait()

                @pl.when(k + 2 < chunks)
                def _():
                    b2 = (b + 2) % _NBUF
                    wait_idx(b2)
                    fire_gather(b2)

                pltpu.sync_copy(rows[b], acc.at[idx_d[b]], add=True)

                @pl.when(k + _NBUF < chunks)
                def _():
                    load_idx(k + _NBUF, b)
            return carry

        lax.fori_loop(0, chunks // _NBUF, tri_body, 0)
        for kk in range(chunks - chunks % _NBUF, chunks):
            b = kk % _NBUF
            pltpu.make_async_copy(h_lo.at[idx_s[b]], rows[b],
                                  sems[b]).wait()
            if kk + 2 < chunks:
                b2 = (b + 2) % _NBUF
                wait_idx(b2)
                fire_gather(b2)
            pltpu.sync_copy(rows[b], acc.at[idx_d[b]], add=True)

        if etail:
            tbase = ebase + chunks * _EBLK
            pltpu.sync_copy(src.at[pl.ds(tbase, etail)], tidx_s.at[pl.ds(0, etail)])
            pltpu.sync_copy(dst.at[pl.ds(tbase, etail)], tidx_d.at[pl.ds(0, etail)])

            @pl.when(c == 0)
            def _():
                pltpu.async_copy(h_lo.at[tidx_s], rows0.at[pl.ds(0, max(etail, 8))],
                                 sem0).wait()

            @pl.when(c == 1)
            def _():
                pltpu.async_copy(h_hi.at[tidx_s], rows0.at[pl.ds(0, max(etail, 8))],
                                 sem1).wait()

            pltpu.sync_copy(rows0.at[pl.ds(0, max(etail, 8))], acc.at[tidx_d],
                            add=True)
        plsc.subcore_barrier()
        pltpu.sync_copy(
            acc.at[pl.ds(s * rows_per_tile, rows_per_tile)],
            out.at[pl.ds(c * n + s * rows_per_tile, rows_per_tile)],
        )
        if tail:
            @pl.when(s == _NS - 1)
            def _():
                pltpu.sync_copy(
                    acc.at[pl.ds(_NS * rows_per_tile, tail)],
                    out.at[pl.ds(c * n + _NS * rows_per_tile, tail)],
                )

    return agg


def kernel(x, edge_index, batch, W_enc, b_enc, eps1, W1_1, b1_1, W2_1, b2_1,
           gamma1, beta1, eps2, W1_2, b1_2, W2_2, b2_2, gamma2, beta2):
    n, f = x.shape
    d = W_enc.shape[1]
    dh = d // 2
    e = edge_index.shape[1]

    zeros = jnp.zeros((n // _NS, dh), jnp.float32)
    eps1_ = jnp.reshape(eps1, (1, 1))
    eps2_ = jnp.reshape(eps2, (1, 1))
    b_enc_ = jnp.reshape(b_enc, (1, d))
    b1_1_ = jnp.reshape(b1_1, (1, 2 * d))
    b2_1_ = jnp.reshape(b2_1, (1, d))
    gamma1_ = jnp.reshape(gamma1, (1, d))
    beta1_ = jnp.reshape(beta1, (1, d))
    b1_2_ = jnp.reshape(b1_2, (1, 2 * d))
    b2_2_ = jnp.reshape(b2_2, (1, d))
    gamma2_ = jnp.reshape(gamma2, (1, d))
    beta2_ = jnp.reshape(beta2, (1, d))
    batch_ = jnp.reshape(batch, (1, n))

    agg_fn = _make_agg(n, e, dh)

    edges_flat = jnp.ravel(edge_index)

    h0lo, h0hi = _encoder(x, W_enc, b_enc_)
    agg1 = agg_fn(h0lo, h0hi, edges_flat, zeros)
    h1lo, h1hi = _mlp_bn(h0lo, h0hi, agg1, eps1_, W1_1, b1_1_, W2_1, b2_1_,
                         gamma1_, beta1_)
    agg2 = agg_fn(h1lo, h1hi, edges_flat, zeros)
    hg, vg = _mlp_bn_pool(h1lo, h1hi, agg2, eps2_, W1_2, b1_2_, W2_2, b2_2_,
                          gamma2_, beta2_, batch_)
    return (hg, vg)
